# Initial kernel scaffold; baseline (speedup 1.0000x reference)
import jax
import jax.numpy as jnp
from jax.experimental import pallas as pl


def kernel(values, cu_seqlens, seqlens, W, b):
    raise NotImplementedError("write your pallas kernel here")


# speedup vs baseline: 2615.1337x; 2615.1337x over previous
import functools

import jax
import jax.numpy as jnp
from jax import lax
from jax.experimental import pallas as pl
from jax.experimental.pallas import tpu as pltpu
from jax.experimental.pallas import tpu_sc as plsc

L = 16
TILE = 4000


def _build(B, total, maxlen):
    NT = B // TILE
    info = plsc.get_sparse_core_info()
    NW = info.num_cores * info.num_subcores
    ITERS = -(-NT // NW)

    CUSZ = TILE + 16 + ((B + 1 - (TILE + 16)) % 8)
    CU_CLAMP = B + 1 - CUSZ
    VSZ_BASE = maxlen * TILE + 24
    VSZ = VSZ_BASE + ((total - VSZ_BASE) % 8)
    V_CLAMP = total - VSZ
    NGRP = TILE // L

    mesh = plsc.VectorSubcoreMesh(core_axis_name="c", subcore_axis_name="s")

    @functools.partial(
        pl.kernel,
        mesh=mesh,
        out_type=jax.ShapeDtypeStruct((B,), jnp.float32),
        compiler_params=pltpu.CompilerParams(needs_layout_passes=False),
        scratch_types=[
            pltpu.VMEM((VSZ,), jnp.float32),
            pltpu.VMEM((CUSZ,), jnp.int32),
            pltpu.VMEM((TILE,), jnp.float32),
            pltpu.VMEM((maxlen + 1, L), jnp.float32),
        ],
    )
    def sck(vals_hbm, cu_hbm, waux_hbm, out_hbm, valv, cuv, outv, wv):
        wid = lax.axis_index("s") * info.num_cores + lax.axis_index("c")
        pltpu.sync_copy(waux_hbm, wv)
        wrows = [wv[j] for j in range(maxlen)]
        bv = wv[maxlen]

        def tile_body(i, carry):
            t = wid + i * NW

            @pl.when(t < NT)
            def _():
                r0 = t * TILE
                custart = jnp.minimum(r0, CU_CLAMP)
                delta = r0 - custart
                pltpu.sync_copy(cu_hbm.at[pl.ds(custart, CUSZ)], cuv)
                vstart = cuv[pl.ds(delta, L)][0]
                vbase = jnp.minimum((vstart // 8) * 8, V_CLAMP)
                pltpu.sync_copy(vals_hbm.at[pl.ds(vbase, VSZ)], valv)

                def group(g, c):
                    off = delta + g * L
                    cur = cuv[pl.ds(off, L)]
                    nxt = cuv[pl.ds(off + 1, L)]
                    ln = nxt - cur
                    rel = cur - vbase
                    acc = bv
                    for j in range(maxlen):
                        idx = jnp.minimum(rel + j, VSZ - 1)
                        gj = plsc.load_gather(valv, [idx])
                        acc = acc + jnp.where(ln > j, gj * wrows[j], 0.0)
                    outv[pl.ds(g * L, L)] = acc
                    return c

                lax.fori_loop(0, NGRP, group, 0)
                pltpu.sync_copy(outv, out_hbm.at[pl.ds(r0, TILE)])

            return carry

        lax.fori_loop(0, ITERS, tile_body, 0)

    return sck


def kernel(values, cu_seqlens, seqlens, W, b):
    B = cu_seqlens.shape[0] - 1
    maxlen = W.shape[0]
    total = values.shape[0]
    min_total = maxlen * TILE + 24 + 16
    if total < min_total:
        values = jnp.pad(values, (0, min_total - total))
        total = min_total
    waux = jnp.concatenate(
        [
            jnp.broadcast_to(W.reshape(maxlen, 1), (maxlen, L)),
            jnp.broadcast_to(b.reshape(1, 1), (1, L)),
        ],
        axis=0,
    ).astype(jnp.float32)
    out = _build(B, total, maxlen)(values, cu_seqlens, waux)
    return out.reshape(B, 1)


# speedup vs baseline: 4067.0007x; 1.5552x over previous
import functools

import jax
import jax.numpy as jnp
from jax import lax
from jax.experimental import pallas as pl
from jax.experimental.pallas import tpu as pltpu
from jax.experimental.pallas import tpu_sc as plsc

L = 16
STAGES = 15


def _build(B, total, maxlen):
    info = plsc.get_sparse_core_info()
    NW = info.num_cores * info.num_subcores
    LO = (B // NW) // L * L
    REM = (B - LO * NW) // L
    HI = LO + (L if REM else 0)
    CH = L * (-(-HI // (STAGES * L)))
    NGRP = CH // L

    CUSZ = CH + 16 + ((B + 1 - (CH + 16)) % 8)
    CU_CLAMP = B + 1 - CUSZ
    VSZ_BASE = maxlen * CH + 24
    VSZ = VSZ_BASE + ((total - VSZ_BASE) % 8)
    V_CLAMP = total - VSZ

    mesh = plsc.VectorSubcoreMesh(core_axis_name="c", subcore_axis_name="s")

    @functools.partial(
        pl.kernel,
        mesh=mesh,
        out_type=jax.ShapeDtypeStruct((B,), jnp.float32),
        compiler_params=pltpu.CompilerParams(needs_layout_passes=False),
        scratch_types=[
            pltpu.VMEM((VSZ,), jnp.float32),
            pltpu.VMEM((VSZ,), jnp.float32),
            pltpu.VMEM((CUSZ,), jnp.int32),
            pltpu.VMEM((CUSZ,), jnp.int32),
            pltpu.VMEM((CUSZ,), jnp.int32),
            pltpu.VMEM((CH,), jnp.float32),
            pltpu.VMEM((CH,), jnp.float32),
            pltpu.VMEM((maxlen + 1, L), jnp.float32),
            pltpu.SemaphoreType.DMA,
            pltpu.SemaphoreType.DMA,
            pltpu.SemaphoreType.DMA,
            pltpu.SemaphoreType.DMA,
            pltpu.SemaphoreType.DMA,
            pltpu.SemaphoreType.DMA,
            pltpu.SemaphoreType.DMA,
        ],
    )
    def sck(vals_hbm, cu_hbm, waux_hbm, out_hbm,
            valv0, valv1, cuv0, cuv1, cuv2, outv0, outv1, wv,
            sv0, sv1, sc0, sc1, sc2, so0, so1):
        wid = lax.axis_index("s") * info.num_cores + lax.axis_index("c")
        pltpu.sync_copy(waux_hbm, wv)
        wrows = [wv[j] for j in range(maxlen)]
        bv = wv[maxlen]

        valvs, vsems = [valv0, valv1], [sv0, sv1]
        cuvs, csems = [cuv0, cuv1, cuv2], [sc0, sc1, sc2]
        outvs, osems = [outv0, outv1], [so0, so1]

        row0 = LO * wid + L * jnp.maximum(wid - (NW - REM), 0)
        rows_w = LO + L * (wid >= NW - REM).astype(jnp.int32)

        starts, deltas, vbases = {}, {}, {}

        def issue_cu(k):
            start = row0 + jnp.minimum(k * CH, rows_w - CH)
            cst = jnp.minimum(start, CU_CLAMP)
            starts[k], deltas[k] = start, start - cst
            return pltpu.async_copy(
                cu_hbm.at[pl.ds(cst, CUSZ)], cuvs[k % 3], csems[k % 3])

        def issue_val(k):
            vstart = cuvs[k % 3][pl.ds(deltas[k], L)][0]
            vbase = jnp.minimum((vstart // 8) * 8, V_CLAMP)
            vbases[k] = vbase
            return pltpu.async_copy(
                vals_hbm.at[pl.ds(vbase, VSZ)], valvs[k % 2], vsems[k % 2])

        def compute(k):
            cuv, valv, outv = cuvs[k % 3], valvs[k % 2], outvs[k % 2]
            delta, vbase = deltas[k], vbases[k]

            def group(g, c):
                off = delta + g * L
                cur = cuv[pl.ds(off, L)]
                nxt = cuv[pl.ds(off + 1, L)]
                ln = nxt - cur
                rel = cur - vbase
                acc = bv
                for j in range(maxlen):
                    idx = jnp.minimum(rel + j, VSZ - 1)
                    gj = plsc.load_gather(valv, [idx])
                    acc = acc + jnp.where(ln > j, gj * wrows[j], 0.0)
                outv[pl.ds(g * L, L)] = acc
                return c

            lax.fori_loop(0, NGRP, group, 0)
            return pltpu.async_copy(
                outv, out_hbm.at[pl.ds(starts[k], CH)], osems[k % 2])

        h_cu, h_val, h_out = {}, {}, [None, None]
        h_cu[0] = issue_cu(0)
        h_cu[0].wait()
        h_val[0] = issue_val(0)
        if STAGES > 1:
            h_cu[1] = issue_cu(1)
        for k in range(STAGES):
            if k + 1 < STAGES:
                h_cu[k + 1].wait()
                h_val[k + 1] = issue_val(k + 1)
            if k + 2 < STAGES:
                h_cu[k + 2] = issue_cu(k + 2)
            h_val[k].wait()
            if h_out[k % 2] is not None:
                h_out[k % 2].wait()
            h_out[k % 2] = compute(k)
        for h in h_out:
            if h is not None:
                h.wait()

    return sck


def kernel(values, cu_seqlens, seqlens, W, b):
    B = cu_seqlens.shape[0] - 1
    maxlen = W.shape[0]
    total = values.shape[0]
    min_total = maxlen * (B // (2 * STAGES) + 256) + 256
    if total < min_total:
        values = jnp.pad(values, (0, min_total - total))
        total = min_total
    waux = jnp.concatenate(
        [
            jnp.broadcast_to(W.reshape(maxlen, 1), (maxlen, L)),
            jnp.broadcast_to(b.reshape(1, 1), (1, L)),
        ],
        axis=0,
    ).astype(jnp.float32)
    out = _build(B, total, maxlen)(values, cu_seqlens, waux)
    return out.reshape(B, 1)


# speedup vs baseline: 4128.7828x; 1.0152x over previous
import functools

import jax
import jax.numpy as jnp
from jax import lax
from jax.experimental import pallas as pl
from jax.experimental.pallas import tpu as pltpu
from jax.experimental.pallas import tpu_sc as plsc

L = 16
STAGES = 15


def _build(B, total, maxlen):
    info = plsc.get_sparse_core_info()
    NW = info.num_cores * info.num_subcores
    LO = (B // NW) // L * L
    REM = (B - LO * NW) // L
    HI = LO + (L if REM else 0)
    CH = 2 * L * (-(-HI // (STAGES * 2 * L)))
    NGRP = CH // L

    CUSZ = CH + 16 + ((B + 1 - (CH + 16)) % 8)
    CU_CLAMP = B + 1 - CUSZ
    VSZ_BASE = maxlen * CH + 24
    VSZ = VSZ_BASE + ((total - VSZ_BASE) % 8)
    V_CLAMP = total - VSZ

    mesh = plsc.VectorSubcoreMesh(core_axis_name="c", subcore_axis_name="s")

    @functools.partial(
        pl.kernel,
        mesh=mesh,
        out_type=jax.ShapeDtypeStruct((B,), jnp.float32),
        compiler_params=pltpu.CompilerParams(needs_layout_passes=False),
        scratch_types=[
            pltpu.VMEM((VSZ,), jnp.float32),
            pltpu.VMEM((VSZ,), jnp.float32),
            pltpu.VMEM((CUSZ,), jnp.int32),
            pltpu.VMEM((CUSZ,), jnp.int32),
            pltpu.VMEM((CUSZ,), jnp.int32),
            pltpu.VMEM((CH,), jnp.float32),
            pltpu.VMEM((CH,), jnp.float32),
            pltpu.VMEM((maxlen + 1, L), jnp.float32),
            pltpu.SemaphoreType.DMA,
            pltpu.SemaphoreType.DMA,
            pltpu.SemaphoreType.DMA,
            pltpu.SemaphoreType.DMA,
            pltpu.SemaphoreType.DMA,
            pltpu.SemaphoreType.DMA,
            pltpu.SemaphoreType.DMA,
        ],
    )
    def sck(vals_hbm, cu_hbm, waux_hbm, out_hbm,
            valv0, valv1, cuv0, cuv1, cuv2, outv0, outv1, wv,
            sv0, sv1, sc0, sc1, sc2, so0, so1):
        wid = lax.axis_index("s") * info.num_cores + lax.axis_index("c")
        pltpu.sync_copy(waux_hbm, wv)
        wrows = [wv[j] for j in range(maxlen)]
        bv = wv[maxlen]

        valvs, vsems = [valv0, valv1], [sv0, sv1]
        cuvs, csems = [cuv0, cuv1, cuv2], [sc0, sc1, sc2]
        outvs, osems = [outv0, outv1], [so0, so1]

        row0 = LO * wid + L * jnp.maximum(wid - (NW - REM), 0)
        rows_w = LO + L * (wid >= NW - REM).astype(jnp.int32)

        starts, deltas, vbases = {}, {}, {}

        def issue_cu(k):
            start = row0 + jnp.minimum(k * CH, rows_w - CH)
            cst = jnp.minimum(start, CU_CLAMP)
            starts[k], deltas[k] = start, start - cst
            return pltpu.async_copy(
                cu_hbm.at[pl.ds(cst, CUSZ)], cuvs[k % 3], csems[k % 3])

        def issue_val(k):
            vstart = cuvs[k % 3][pl.ds(deltas[k], L)][0]
            vbase = jnp.minimum((vstart // 8) * 8, V_CLAMP)
            vbases[k] = vbase
            return pltpu.async_copy(
                vals_hbm.at[pl.ds(vbase, VSZ)], valvs[k % 2], vsems[k % 2])

        def compute(k):
            cuv, valv, outv = cuvs[k % 3], valvs[k % 2], outvs[k % 2]
            delta, vbase = deltas[k], vbases[k]

            def group(g, c):
                for h in range(2):
                    off = delta + g * (2 * L) + h * L
                    cur = cuv[pl.ds(off, L)]
                    nxt = cuv[pl.ds(off + 1, L)]
                    ln = nxt - cur
                    rel = cur - vbase
                    acc = bv
                    for j in range(maxlen):
                        m = ln > j
                        gj = plsc.load_gather(valv, [rel + j], mask=m)
                        acc = acc + jnp.where(m, gj * wrows[j], 0.0)
                    outv[pl.ds(g * (2 * L) + h * L, L)] = acc
                return c

            lax.fori_loop(0, NGRP // 2, group, 0)
            return pltpu.async_copy(
                outv, out_hbm.at[pl.ds(starts[k], CH)], osems[k % 2])

        h_cu, h_val, h_out = {}, {}, [None, None]
        h_cu[0] = issue_cu(0)
        h_cu[0].wait()
        h_val[0] = issue_val(0)
        if STAGES > 1:
            h_cu[1] = issue_cu(1)
        for k in range(STAGES):
            if k + 1 < STAGES:
                h_cu[k + 1].wait()
                h_val[k + 1] = issue_val(k + 1)
            if k + 2 < STAGES:
                h_cu[k + 2] = issue_cu(k + 2)
            h_val[k].wait()
            if h_out[k % 2] is not None:
                h_out[k % 2].wait()
            h_out[k % 2] = compute(k)
        for h in h_out:
            if h is not None:
                h.wait()

    return sck


def kernel(values, cu_seqlens, seqlens, W, b):
    B = cu_seqlens.shape[0] - 1
    maxlen = W.shape[0]
    total = values.shape[0]
    min_total = maxlen * (B // (2 * STAGES) + 256) + 256
    if total < min_total:
        values = jnp.pad(values, (0, min_total - total))
        total = min_total
    waux = jnp.concatenate(
        [
            jnp.broadcast_to(W.reshape(maxlen, 1), (maxlen, L)),
            jnp.broadcast_to(b.reshape(1, 1), (1, L)),
        ],
        axis=0,
    ).astype(jnp.float32)
    out = _build(B, total, maxlen)(values, cu_seqlens, waux)
    return out.reshape(B, 1)


# speedup vs baseline: 4193.4145x; 1.0157x over previous
import functools

import jax
import jax.numpy as jnp
from jax import lax
from jax.experimental import pallas as pl
from jax.experimental.pallas import tpu as pltpu
from jax.experimental.pallas import tpu_sc as plsc

L = 16
STAGES = 15


def _build(B, total, maxlen):
    info = plsc.get_sparse_core_info()
    NW = info.num_cores * info.num_subcores
    LO = (B // NW) // L * L
    REM = (B - LO * NW) // L
    HI = LO + (L if REM else 0)
    CH = 2 * L * (-(-HI // (STAGES * 2 * L)))
    NGRP = CH // L

    CUSZ = CH + 8
    VSZ_BASE = maxlen * CH + 24
    VSZ = VSZ_BASE + ((total - VSZ_BASE) % 8)
    V_CLAMP = total - VSZ

    mesh = plsc.VectorSubcoreMesh(core_axis_name="c", subcore_axis_name="s")

    @functools.partial(
        pl.kernel,
        mesh=mesh,
        out_type=jax.ShapeDtypeStruct((B,), jnp.float32),
        compiler_params=pltpu.CompilerParams(needs_layout_passes=False),
        scratch_types=[
            pltpu.VMEM((VSZ,), jnp.float32),
            pltpu.VMEM((VSZ,), jnp.float32),
            pltpu.VMEM((CUSZ,), jnp.int32),
            pltpu.VMEM((CUSZ,), jnp.int32),
            pltpu.VMEM((CUSZ,), jnp.int32),
            pltpu.VMEM((CH,), jnp.float32),
            pltpu.VMEM((CH,), jnp.float32),
            pltpu.VMEM((maxlen + 1, L), jnp.float32),
            pltpu.SemaphoreType.DMA,
            pltpu.SemaphoreType.DMA,
            pltpu.SemaphoreType.DMA,
            pltpu.SemaphoreType.DMA,
            pltpu.SemaphoreType.DMA,
            pltpu.SemaphoreType.DMA,
            pltpu.SemaphoreType.DMA,
        ],
    )
    def sck(vals_hbm, cu_hbm, waux_hbm, out_hbm,
            valv0, valv1, cuv0, cuv1, cuv2, outv0, outv1, wv,
            sv0, sv1, sc0, sc1, sc2, so0, so1):
        wid = lax.axis_index("s") * info.num_cores + lax.axis_index("c")
        pltpu.sync_copy(waux_hbm, wv)
        wrows = [wv[j] for j in range(maxlen)]
        bv = wv[maxlen]

        valvs, vsems = [valv0, valv1], [sv0, sv1]
        cuvs, csems = [cuv0, cuv1, cuv2], [sc0, sc1, sc2]
        outvs, osems = [outv0, outv1], [so0, so1]

        row0 = LO * wid + L * jnp.maximum(wid - (NW - REM), 0)
        rows_w = LO + L * (wid >= NW - REM).astype(jnp.int32)

        starts, vbases = {}, {}

        def issue_cu(k):
            start = row0 + jnp.minimum(k * CH, rows_w - CH)
            starts[k] = start
            return pltpu.async_copy(
                cu_hbm.at[pl.ds(start, CH + 1)],
                cuvs[k % 3].at[pl.ds(0, CH + 1)], csems[k % 3])

        def issue_val(k):
            vstart = cuvs[k % 3][pl.ds(0, L)][0]
            vbase = jnp.minimum((vstart // 8) * 8, V_CLAMP)
            vbases[k] = vbase
            return pltpu.async_copy(
                vals_hbm.at[pl.ds(vbase, VSZ)], valvs[k % 2], vsems[k % 2])

        def compute(k):
            cuv, valv, outv = cuvs[k % 3], valvs[k % 2], outvs[k % 2]
            vbase = vbases[k]

            def group(g, c):
                for h in range(2):
                    off = g * (2 * L) + h * L
                    cur = cuv[pl.ds(off, L)]
                    nxt = cuv[pl.ds(off + 1, L)]
                    ln = nxt - cur
                    rel = cur - vbase
                    acc = bv
                    for j in range(maxlen):
                        m = ln > j
                        gj = plsc.load_gather(valv, [rel + j], mask=m)
                        acc = acc + jnp.where(m, gj * wrows[j], 0.0)
                    outv[pl.ds(g * (2 * L) + h * L, L)] = acc
                return c

            lax.fori_loop(0, NGRP // 2, group, 0)
            return pltpu.async_copy(
                outv, out_hbm.at[pl.ds(starts[k], CH)], osems[k % 2])

        h_cu, h_val, h_out = {}, {}, [None, None]
        h_cu[0] = issue_cu(0)
        h_cu[0].wait()
        h_val[0] = issue_val(0)
        if STAGES > 1:
            h_cu[1] = issue_cu(1)
        for k in range(STAGES):
            if k + 1 < STAGES:
                h_cu[k + 1].wait()
                h_val[k + 1] = issue_val(k + 1)
            if k + 2 < STAGES:
                h_cu[k + 2] = issue_cu(k + 2)
            h_val[k].wait()
            if h_out[k % 2] is not None:
                h_out[k % 2].wait()
            h_out[k % 2] = compute(k)
        for h in h_out:
            if h is not None:
                h.wait()

    return sck


def kernel(values, cu_seqlens, seqlens, W, b):
    B = cu_seqlens.shape[0] - 1
    maxlen = W.shape[0]
    total = values.shape[0]
    min_total = maxlen * (B // (2 * STAGES) + 256) + 256
    if total < min_total:
        values = jnp.pad(values, (0, min_total - total))
        total = min_total
    waux = jnp.concatenate(
        [
            jnp.broadcast_to(W.reshape(maxlen, 1), (maxlen, L)),
            jnp.broadcast_to(b.reshape(1, 1), (1, L)),
        ],
        axis=0,
    ).astype(jnp.float32)
    out = _build(B, total, maxlen)(values, cu_seqlens, waux)
    return out.reshape(B, 1)


# speedup vs baseline: 5046.4612x; 1.2034x over previous
import functools

import jax
import jax.numpy as jnp
from jax import lax
from jax.experimental import pallas as pl
from jax.experimental.pallas import tpu as pltpu
from jax.experimental.pallas import tpu_sc as plsc

L = 16
STAGES = 15


def _build(B, total, maxlen):
    info = plsc.get_sparse_core_info()
    NW = info.num_cores * info.num_subcores
    LO = (B // NW) // L * L
    REM = (B - LO * NW) // L
    HI = LO + (L if REM else 0)
    CH = 2 * L * (-(-HI // (STAGES * 2 * L)))
    NGRP = CH // L

    CUSZ = CH + 8
    VSZ_BASE = maxlen * CH + 24
    VSZ = VSZ_BASE + ((total - VSZ_BASE) % 8)
    V_CLAMP = total - VSZ

    mesh = plsc.VectorSubcoreMesh(core_axis_name="c", subcore_axis_name="s")

    @functools.partial(
        pl.kernel,
        mesh=mesh,
        out_type=jax.ShapeDtypeStruct((B,), jnp.float32),
        compiler_params=pltpu.CompilerParams(needs_layout_passes=False),
        scratch_types=[
            pltpu.VMEM((VSZ,), jnp.float32),
            pltpu.VMEM((VSZ,), jnp.float32),
            pltpu.VMEM((CUSZ,), jnp.int32),
            pltpu.VMEM((CUSZ,), jnp.int32),
            pltpu.VMEM((CUSZ,), jnp.int32),
            pltpu.VMEM((CH,), jnp.float32),
            pltpu.VMEM((CH,), jnp.float32),
            pltpu.VMEM((maxlen + 1, L), jnp.float32),
            pltpu.SemaphoreType.DMA,
            pltpu.SemaphoreType.DMA,
            pltpu.SemaphoreType.DMA,
            pltpu.SemaphoreType.DMA,
            pltpu.SemaphoreType.DMA,
            pltpu.SemaphoreType.DMA,
            pltpu.SemaphoreType.DMA,
        ],
    )
    def sck(vals_hbm, cu_hbm, waux_hbm, out_hbm,
            valv0, valv1, cuv0, cuv1, cuv2, outv0, outv1, wv,
            sv0, sv1, sc0, sc1, sc2, so0, so1):
        wid = lax.axis_index("s") * info.num_cores + lax.axis_index("c")
        pltpu.sync_copy(waux_hbm, wv)
        wrows = [wv[j] for j in range(maxlen)]
        bv = wv[maxlen]

        valvs, vsems = [valv0, valv1], [sv0, sv1]
        cuvs, csems = [cuv0, cuv1, cuv2], [sc0, sc1, sc2]
        outvs, osems = [outv0, outv1], [so0, so1]

        row0 = LO * wid + L * jnp.maximum(wid - (NW - REM), 0)
        rows_w = LO + L * (wid >= NW - REM).astype(jnp.int32)

        starts, vbases = {}, {}

        def issue_cu(k):
            start = row0 + jnp.minimum(k * CH, rows_w - CH)
            starts[k] = start
            return pltpu.async_copy(
                cu_hbm.at[pl.ds(start, CH + 1)],
                cuvs[k % 3].at[pl.ds(0, CH + 1)], csems[k % 3])

        def issue_val(k):
            vstart = cuvs[k % 3][pl.ds(0, L)][0]
            vbase = jnp.minimum((vstart // 8) * 8, V_CLAMP)
            vbases[k] = vbase
            return pltpu.async_copy(
                vals_hbm.at[pl.ds(vbase, VSZ)], valvs[k % 2], vsems[k % 2])

        def compute(k):
            cuv, valv, outv = cuvs[k % 3], valvs[k % 2], outvs[k % 2]
            vbase = vbases[k]

            @plsc.parallel_loop(0, NGRP, 1, unroll=4)
            def group(g):
                off = g * L
                cur = cuv[pl.ds(off, L)]
                nxt = cuv[pl.ds(off + 1, L)]
                ln = nxt - cur
                rel = cur - vbase
                acc = bv
                for j in range(maxlen):
                    m = ln > j
                    gj = plsc.load_gather(valv, [rel + j], mask=m)
                    acc = acc + jnp.where(m, gj * wrows[j], 0.0)
                outv[pl.ds(off, L)] = acc
            return pltpu.async_copy(
                outv, out_hbm.at[pl.ds(starts[k], CH)], osems[k % 2])

        h_cu, h_val, h_out = {}, {}, [None, None]
        h_cu[0] = issue_cu(0)
        h_cu[0].wait()
        h_val[0] = issue_val(0)
        if STAGES > 1:
            h_cu[1] = issue_cu(1)
        for k in range(STAGES):
            if k + 1 < STAGES:
                h_cu[k + 1].wait()
                h_val[k + 1] = issue_val(k + 1)
            if k + 2 < STAGES:
                h_cu[k + 2] = issue_cu(k + 2)
            h_val[k].wait()
            if h_out[k % 2] is not None:
                h_out[k % 2].wait()
            h_out[k % 2] = compute(k)
        for h in h_out:
            if h is not None:
                h.wait()

    return sck


def kernel(values, cu_seqlens, seqlens, W, b):
    B = cu_seqlens.shape[0] - 1
    maxlen = W.shape[0]
    total = values.shape[0]
    min_total = maxlen * (B // (2 * STAGES) + 256) + 256
    if total < min_total:
        values = jnp.pad(values, (0, min_total - total))
        total = min_total
    waux = jnp.concatenate(
        [
            jnp.broadcast_to(W.reshape(maxlen, 1), (maxlen, L)),
            jnp.broadcast_to(b.reshape(1, 1), (1, L)),
        ],
        axis=0,
    ).astype(jnp.float32)
    out = _build(B, total, maxlen)(values, cu_seqlens, waux)
    return out.reshape(B, 1)
